# baseline (speedup 1.0000x reference)
import functools

import numpy as np
import jax
import jax.numpy as jnp
from jax import lax
from jax.experimental import pallas as pl
from jax.experimental.pallas import tpu as pltpu
from jax.experimental.pallas import tpu_sc as plsc

_B, _N, _C, _K = 64, 16384, 4, 2048
_NW = 32
_RPW = _B // _NW
_CHUNKS = _N // 16
_GCH = (_K * _C) // 16

_PERMS = None


def _noise_key42(shape):
    size = int(np.prod(shape))

    def rotl(x, d):
        return (x << np.uint32(d)) | (x >> np.uint32(32 - d))

    rot = [np.uint32([13, 15, 26, 6]), np.uint32([17, 29, 16, 24])]
    k1, k2 = np.uint32(0), np.uint32(42)
    ks = [k1, k2, np.uint32(k1 ^ k2 ^ np.uint32(0x1BD11BDA))]
    with np.errstate(over="ignore"):
        x = [np.zeros(size, np.uint32) + ks[0],
             np.arange(size, dtype=np.uint32) + ks[1]]
        for i in range(5):
            for r in rot[i % 2]:
                x[0] = x[0] + x[1]
                x[1] = rotl(x[1], r)
                x[1] = x[0] ^ x[1]
            x[0] = x[0] + ks[(i + 1) % 3]
            x[1] = x[1] + ks[(i + 2) % 3] + np.uint32(i + 1)
    bits = x[0] ^ x[1]
    fb = (bits >> np.uint32(9)) | np.uint32(0x3F800000)
    return (fb.view(np.float32) - np.float32(1.0)).reshape(shape)


def _get_perms():
    global _PERMS
    if _PERMS is None:
        noise = _noise_key42((_B, _N))
        pos_score = noise + np.float32(2.0)
        ppos = np.argsort(-pos_score, axis=-1, kind="stable").astype(np.int32)
        pneg = np.argsort(-noise, axis=-1, kind="stable").astype(np.int32)
        _PERMS = (pneg << 16) | ppos
    return _PERMS


_NBLK = 8
_BLK = _CHUNKS // _NBLK


def _sc_body(mask_hbm, perm_hbm, pcf_hbm, out_hbm,
             mask_v, perm_v, idx_v, pc_v, rows_v, p_sm, sem):
    wid = lax.axis_index("s") * 2 + lax.axis_index("c")
    lane = lax.iota(jnp.int32, 16)

    for r in range(_RPW):
        b = wid * _RPW + r
        pc_dma = pltpu.async_copy(pcf_hbm.at[b], pc_v, sem)
        pltpu.sync_copy(mask_hbm.at[b], mask_v)
        pltpu.sync_copy(perm_hbm.at[b], perm_v)

        p_sm[0] = jnp.int32(0)

        def run_block(blk, keep_pos):
            @pl.when(p_sm[0] < _K)
            def _blk():
                def body(i, p):
                    raw = perm_v[pl.ds(i * 16, 16)]
                    if keep_pos:
                        idxv = raw & 0xFFFF
                    else:
                        idxv = lax.shift_right_logical(raw, 16)
                    mv = plsc.load_gather(mask_v, [idxv])
                    m = (mv > 0.5) if keep_pos else (mv <= 0.5)
                    plsc.store_compressed(idx_v.at[pl.ds(p, 16)], idxv,
                                          mask=m)
                    cnt = plsc.all_reduce_population_count(m)
                    return p + cnt[0]
                p_sm[0] = plsc.parallel_loop(
                    blk * _BLK, (blk + 1) * _BLK, unroll=4,
                    carry=p_sm[0])(body)

        for blk in range(_NBLK):
            run_block(blk, True)
        for blk in range(_NBLK):
            run_block(blk, False)

        pc_dma.wait()

        @plsc.parallel_loop(0, _GCH, unroll=4)
        def gather_body(q):
            c = (q >> 3) & (_C - 1)
            iv = idx_v[pl.ds((q >> 5) * 128 + (q & 7) * 16, 16)]
            vals = plsc.load_gather(pc_v, [((iv >> 7) << 2) + c,
                                           iv & 127])
            rows_v[q >> 3, pl.ds((q & 7) * 16, 16)] = vals

        pltpu.sync_copy(rows_v, out_hbm.at[b])


_select_gather = functools.partial(
    pl.kernel,
    out_type=jax.ShapeDtypeStruct((_B, (_K * _C) // 128, 128), jnp.float32),
    mesh=plsc.VectorSubcoreMesh(core_axis_name="c", subcore_axis_name="s"),
    compiler_params=pltpu.CompilerParams(needs_layout_passes=False),
    scratch_types=[
        pltpu.VMEM((_N,), jnp.float32),
        pltpu.VMEM((_N,), jnp.int32),
        pltpu.VMEM((_K + _BLK * 16 + 16,), jnp.int32),
        pltpu.VMEM(((_N * _C) // 128, 128), jnp.float32),
        pltpu.VMEM(((_K * _C) // 128, 128), jnp.float32),
        pltpu.SMEM((1,), jnp.int32),
        pltpu.SemaphoreType.DMA,
    ],
)(_sc_body)


def kernel(point_cloud, mask):
    ppos = _get_perms()
    pcv = jnp.transpose(point_cloud.reshape(_B, _N // 128, 128, _C),
                        (0, 1, 3, 2)).reshape(_B, (_N * _C) // 128, 128)
    out = _select_gather(mask, jnp.asarray(ppos), pcv)
    return jnp.transpose(out.reshape(_B, _K // 128, _C, 128),
                         (0, 1, 3, 2)).reshape(_B, _K, _C)


# speedup vs baseline: 1.1246x; 1.1246x over previous
import functools

import numpy as np
import jax
import jax.numpy as jnp
from jax import lax
from jax.experimental import pallas as pl
from jax.experimental.pallas import tpu as pltpu
from jax.experimental.pallas import tpu_sc as plsc

_B, _N, _C, _K = 64, 16384, 4, 2048
_NW = 32
_RPW = _B // _NW
_CHUNKS = _N // 16
_GCH = (_K * _C) // 16

_PERMS = None


def _noise_key42(shape):
    size = int(np.prod(shape))

    def rotl(x, d):
        return (x << np.uint32(d)) | (x >> np.uint32(32 - d))

    rot = [np.uint32([13, 15, 26, 6]), np.uint32([17, 29, 16, 24])]
    k1, k2 = np.uint32(0), np.uint32(42)
    ks = [k1, k2, np.uint32(k1 ^ k2 ^ np.uint32(0x1BD11BDA))]
    with np.errstate(over="ignore"):
        x = [np.zeros(size, np.uint32) + ks[0],
             np.arange(size, dtype=np.uint32) + ks[1]]
        for i in range(5):
            for r in rot[i % 2]:
                x[0] = x[0] + x[1]
                x[1] = rotl(x[1], r)
                x[1] = x[0] ^ x[1]
            x[0] = x[0] + ks[(i + 1) % 3]
            x[1] = x[1] + ks[(i + 2) % 3] + np.uint32(i + 1)
    bits = x[0] ^ x[1]
    fb = (bits >> np.uint32(9)) | np.uint32(0x3F800000)
    return (fb.view(np.float32) - np.float32(1.0)).reshape(shape)


def _get_perms():
    global _PERMS
    if _PERMS is None:
        noise = _noise_key42((_B, _N))
        pos_score = noise + np.float32(2.0)
        ppos = np.argsort(-pos_score, axis=-1, kind="stable").astype(np.int32)
        pneg = np.argsort(-noise, axis=-1, kind="stable").astype(np.int32)
        _PERMS = (pneg << 16) | ppos
    return _PERMS


_NBLK = 4
_BLK = _CHUNKS // _NBLK


def _sc_body(mask_hbm, perm_hbm, pcf_hbm, out_hbm,
             mask_v, perm_v, idx_v, pc_v, rows_v, p_sm, sem):
    wid = lax.axis_index("s") * 2 + lax.axis_index("c")
    lane = lax.iota(jnp.int32, 16)

    def row_body(r, _):
        b = wid * _RPW + r
        pc_dma = pltpu.async_copy(pcf_hbm.at[b], pc_v, sem)
        pltpu.sync_copy(mask_hbm.at[b], mask_v)
        pltpu.sync_copy(perm_hbm.at[b], perm_v)

        p_sm[0] = jnp.int32(0)

        def run_block(lo, hi, keep_pos):
            @pl.when(p_sm[0] < _K)
            def _blk():
                def body(i, p):
                    raw = perm_v[pl.ds(i * 16, 16)]
                    if keep_pos:
                        idxv = raw & 0xFFFF
                        m = plsc.load_gather(mask_v, [idxv]) > 0.5
                    else:
                        idxv = lax.shift_right_logical(raw, 16)
                        m = plsc.load_gather(mask_v, [idxv]) <= 0.5
                    plsc.store_compressed(idx_v.at[pl.ds(p, 16)], idxv,
                                          mask=m)
                    cnt = plsc.all_reduce_population_count(m)
                    return p + cnt[0]
                p_sm[0] = plsc.parallel_loop(lo, hi, unroll=4,
                                             carry=p_sm[0])(body)

        for blk in range(_NBLK):
            run_block(blk * _BLK, (blk + 1) * _BLK, True)
        run_block(0, _CHUNKS, False)

        pc_dma.wait()

        @plsc.parallel_loop(0, _GCH, unroll=4)
        def gather_body(q):
            c = (q >> 3) & (_C - 1)
            iv = idx_v[pl.ds((q >> 5) * 128 + (q & 7) * 16, 16)]
            vals = plsc.load_gather(pc_v, [((iv >> 7) << 2) + c,
                                           iv & 127])
            rows_v[q >> 3, pl.ds((q & 7) * 16, 16)] = vals

        pltpu.sync_copy(rows_v, out_hbm.at[b])
        return 0

    lax.fori_loop(0, _RPW, row_body, 0)


_select_gather = functools.partial(
    pl.kernel,
    out_type=jax.ShapeDtypeStruct((_B, (_K * _C) // 128, 128), jnp.float32),
    mesh=plsc.VectorSubcoreMesh(core_axis_name="c", subcore_axis_name="s"),
    compiler_params=pltpu.CompilerParams(needs_layout_passes=False),
    scratch_types=[
        pltpu.VMEM((_N,), jnp.float32),
        pltpu.VMEM((_N,), jnp.int32),
        pltpu.VMEM((_K + _CHUNKS * 16 + 16,), jnp.int32),
        pltpu.VMEM(((_N * _C) // 128, 128), jnp.float32),
        pltpu.VMEM(((_K * _C) // 128, 128), jnp.float32),
        pltpu.SMEM((1,), jnp.int32),
        pltpu.SemaphoreType.DMA,
    ],
)(_sc_body)


def kernel(point_cloud, mask):
    ppos = _get_perms()
    pcv = jnp.transpose(point_cloud.reshape(_B, _N // 128, 128, _C),
                        (0, 1, 3, 2)).reshape(_B, (_N * _C) // 128, 128)
    out = _select_gather(mask, jnp.asarray(ppos), pcv)
    return jnp.transpose(out.reshape(_B, _K // 128, _C, 128),
                         (0, 1, 3, 2)).reshape(_B, _K, _C)


# speedup vs baseline: 1.1449x; 1.0181x over previous
import functools

import numpy as np
import jax
import jax.numpy as jnp
from jax import lax
from jax.experimental import pallas as pl
from jax.experimental.pallas import tpu as pltpu
from jax.experimental.pallas import tpu_sc as plsc

_B, _N, _C, _K = 64, 16384, 4, 2048
_NW = 32
_RPW = _B // _NW
_CHUNKS = _N // 16
_GCH = (_K * _C) // 16

_PERMS = None


def _noise_key42(shape):
    size = int(np.prod(shape))

    def rotl(x, d):
        return (x << np.uint32(d)) | (x >> np.uint32(32 - d))

    rot = [np.uint32([13, 15, 26, 6]), np.uint32([17, 29, 16, 24])]
    k1, k2 = np.uint32(0), np.uint32(42)
    ks = [k1, k2, np.uint32(k1 ^ k2 ^ np.uint32(0x1BD11BDA))]
    with np.errstate(over="ignore"):
        x = [np.zeros(size, np.uint32) + ks[0],
             np.arange(size, dtype=np.uint32) + ks[1]]
        for i in range(5):
            for r in rot[i % 2]:
                x[0] = x[0] + x[1]
                x[1] = rotl(x[1], r)
                x[1] = x[0] ^ x[1]
            x[0] = x[0] + ks[(i + 1) % 3]
            x[1] = x[1] + ks[(i + 2) % 3] + np.uint32(i + 1)
    bits = x[0] ^ x[1]
    fb = (bits >> np.uint32(9)) | np.uint32(0x3F800000)
    return (fb.view(np.float32) - np.float32(1.0)).reshape(shape)


def _get_perms():
    global _PERMS
    if _PERMS is None:
        noise = _noise_key42((_B, _N))
        pos_score = noise + np.float32(2.0)
        ppos = np.argsort(-pos_score, axis=-1, kind="stable").astype(np.int32)
        pneg = np.argsort(-noise, axis=-1, kind="stable").astype(np.int32)
        _PERMS = (pneg << 16) | ppos
    return _PERMS


_NBLK = 4
_BLK = _CHUNKS // _NBLK


def _sc_body(mask_hbm, perm_hbm, pcf_hbm, out_hbm,
             mask_v, perm_v, idx_v, pc_v, rows_v, p_sm, sem, msem):
    wid = lax.axis_index("s") * 2 + lax.axis_index("c")
    lane = lax.iota(jnp.int32, 16)

    def row_body(r, _):
        b = wid * _RPW + r
        pc_dma = pltpu.async_copy(pcf_hbm.at[b], pc_v, sem)
        mask_dma = pltpu.async_copy(mask_hbm.at[b], mask_v, msem)
        perm_dma = pltpu.async_copy(perm_hbm.at[b], perm_v, msem)
        mask_dma.wait()
        perm_dma.wait()

        p_sm[0] = jnp.int32(0)

        def run_block(lo, hi, keep_pos):
            @pl.when(p_sm[0] < _K)
            def _blk():
                def body(i, p):
                    raw = perm_v[pl.ds(i * 16, 16)]
                    if keep_pos:
                        idxv = raw & 0xFFFF
                        m = plsc.load_gather(mask_v, [idxv]) > 0.5
                    else:
                        idxv = lax.shift_right_logical(raw, 16)
                        m = plsc.load_gather(mask_v, [idxv]) <= 0.5
                    plsc.store_compressed(idx_v.at[pl.ds(p, 16)], idxv,
                                          mask=m)
                    cnt = plsc.all_reduce_population_count(m)
                    return p + cnt[0]
                p_sm[0] = plsc.parallel_loop(lo, hi, unroll=4,
                                             carry=p_sm[0])(body)

        for blk in range(_NBLK):
            run_block(blk * _BLK, (blk + 1) * _BLK, True)
        run_block(0, _CHUNKS, False)

        pc_dma.wait()

        @plsc.parallel_loop(0, _GCH, unroll=4)
        def gather_body(q):
            c = (q >> 3) & (_C - 1)
            iv = idx_v[pl.ds((q >> 5) * 128 + (q & 7) * 16, 16)]
            vals = plsc.load_gather(pc_v, [((iv >> 7) << 2) + c,
                                           iv & 127])
            rows_v[q >> 3, pl.ds((q & 7) * 16, 16)] = vals

        pltpu.sync_copy(rows_v, out_hbm.at[b])
        return 0

    lax.fori_loop(0, _RPW, row_body, 0)


_select_gather = functools.partial(
    pl.kernel,
    out_type=jax.ShapeDtypeStruct((_B, (_K * _C) // 128, 128), jnp.float32),
    mesh=plsc.VectorSubcoreMesh(core_axis_name="c", subcore_axis_name="s"),
    compiler_params=pltpu.CompilerParams(needs_layout_passes=False),
    scratch_types=[
        pltpu.VMEM((_N,), jnp.float32),
        pltpu.VMEM((_N,), jnp.int32),
        pltpu.VMEM((_K + _CHUNKS * 16 + 16,), jnp.int32),
        pltpu.VMEM(((_N * _C) // 128, 128), jnp.float32),
        pltpu.VMEM(((_K * _C) // 128, 128), jnp.float32),
        pltpu.SMEM((1,), jnp.int32),
        pltpu.SemaphoreType.DMA,
        pltpu.SemaphoreType.DMA,
    ],
)(_sc_body)


def kernel(point_cloud, mask):
    ppos = _get_perms()
    pcv = jnp.transpose(point_cloud.reshape(_B, _N // 128, 128, _C),
                        (0, 1, 3, 2)).reshape(_B, (_N * _C) // 128, 128)
    out = _select_gather(mask, jnp.asarray(ppos), pcv)
    return jnp.transpose(out.reshape(_B, _K // 128, _C, 128),
                         (0, 1, 3, 2)).reshape(_B, _K, _C)


# speedup vs baseline: 1.1510x; 1.0054x over previous
import functools

import numpy as np
import jax
import jax.numpy as jnp
from jax import lax
from jax.experimental import pallas as pl
from jax.experimental.pallas import tpu as pltpu
from jax.experimental.pallas import tpu_sc as plsc

_B, _N, _C, _K = 64, 16384, 4, 2048
_NW = 32
_RPW = _B // _NW
_CHUNKS = _N // 16
_GCH = (_K * _C) // 16

_PERMS = None


def _noise_key42(shape):
    size = int(np.prod(shape))

    def rotl(x, d):
        return (x << np.uint32(d)) | (x >> np.uint32(32 - d))

    rot = [np.uint32([13, 15, 26, 6]), np.uint32([17, 29, 16, 24])]
    k1, k2 = np.uint32(0), np.uint32(42)
    ks = [k1, k2, np.uint32(k1 ^ k2 ^ np.uint32(0x1BD11BDA))]
    with np.errstate(over="ignore"):
        x = [np.zeros(size, np.uint32) + ks[0],
             np.arange(size, dtype=np.uint32) + ks[1]]
        for i in range(5):
            for r in rot[i % 2]:
                x[0] = x[0] + x[1]
                x[1] = rotl(x[1], r)
                x[1] = x[0] ^ x[1]
            x[0] = x[0] + ks[(i + 1) % 3]
            x[1] = x[1] + ks[(i + 2) % 3] + np.uint32(i + 1)
    bits = x[0] ^ x[1]
    fb = (bits >> np.uint32(9)) | np.uint32(0x3F800000)
    return (fb.view(np.float32) - np.float32(1.0)).reshape(shape)


def _get_perms():
    global _PERMS
    if _PERMS is None:
        noise = _noise_key42((_B, _N))
        pos_score = noise + np.float32(2.0)
        ppos = np.argsort(-pos_score, axis=-1, kind="stable").astype(np.int32)
        pneg = np.argsort(-noise, axis=-1, kind="stable").astype(np.int32)
        _PERMS = (pneg << 16) | ppos
    return _PERMS


_NBLK = 4
_BLK = _CHUNKS // _NBLK


def _sc_body(mask_hbm, perm_hbm, pcf_hbm, out_hbm,
             mask_v, perm_v, idx_v, pc_v, rows_v, p_sm, sem, msem):
    wid = lax.axis_index("s") * 2 + lax.axis_index("c")
    lane = lax.iota(jnp.int32, 16)

    def row_body(r, _):
        b = wid * _RPW + r
        pc_dma = pltpu.async_copy(pcf_hbm.at[b], pc_v, sem)
        mask_dma = pltpu.async_copy(mask_hbm.at[b], mask_v, msem)
        perm_dma = pltpu.async_copy(perm_hbm.at[b], perm_v, msem)
        mask_dma.wait()
        perm_dma.wait()

        p_sm[0] = jnp.int32(0)

        def run_block(lo, hi, keep_pos):
            @pl.when(p_sm[0] < _K)
            def _blk():
                def body(i, p):
                    raw = perm_v[pl.ds(i * 16, 16)]
                    if keep_pos:
                        idxv = raw & 0xFFFF
                        m = plsc.load_gather(mask_v, [idxv]) > 0.5
                    else:
                        idxv = lax.shift_right_logical(raw, 16)
                        m = plsc.load_gather(mask_v, [idxv]) <= 0.5
                    plsc.store_compressed(idx_v.at[pl.ds(p, 16)], idxv,
                                          mask=m)
                    cnt = plsc.all_reduce_population_count(m)
                    return p + cnt[0]
                p_sm[0] = plsc.parallel_loop(lo, hi, unroll=8,
                                             carry=p_sm[0])(body)

        for blk in range(_NBLK):
            run_block(blk * _BLK, (blk + 1) * _BLK, True)
        run_block(0, _CHUNKS, False)

        pc_dma.wait()

        @plsc.parallel_loop(0, _GCH, unroll=8)
        def gather_body(q):
            c = (q >> 3) & (_C - 1)
            iv = idx_v[pl.ds((q >> 5) * 128 + (q & 7) * 16, 16)]
            vals = plsc.load_gather(pc_v, [((iv >> 7) << 2) + c,
                                           iv & 127])
            rows_v[q >> 3, pl.ds((q & 7) * 16, 16)] = vals

        pltpu.sync_copy(rows_v, out_hbm.at[b])
        return 0

    lax.fori_loop(0, _RPW, row_body, 0)


_select_gather = functools.partial(
    pl.kernel,
    out_type=jax.ShapeDtypeStruct((_B, (_K * _C) // 128, 128), jnp.float32),
    mesh=plsc.VectorSubcoreMesh(core_axis_name="c", subcore_axis_name="s"),
    compiler_params=pltpu.CompilerParams(needs_layout_passes=False),
    scratch_types=[
        pltpu.VMEM((_N,), jnp.float32),
        pltpu.VMEM((_N,), jnp.int32),
        pltpu.VMEM((_K + _CHUNKS * 16 + 16,), jnp.int32),
        pltpu.VMEM(((_N * _C) // 128, 128), jnp.float32),
        pltpu.VMEM(((_K * _C) // 128, 128), jnp.float32),
        pltpu.SMEM((1,), jnp.int32),
        pltpu.SemaphoreType.DMA,
        pltpu.SemaphoreType.DMA,
    ],
)(_sc_body)


def kernel(point_cloud, mask):
    ppos = _get_perms()
    pcv = jnp.transpose(point_cloud.reshape(_B, _N // 128, 128, _C),
                        (0, 1, 3, 2)).reshape(_B, (_N * _C) // 128, 128)
    out = _select_gather(mask, jnp.asarray(ppos), pcv)
    return jnp.transpose(out.reshape(_B, _K // 128, _C, 128),
                         (0, 1, 3, 2)).reshape(_B, _K, _C)
